# Initial kernel scaffold; baseline (speedup 1.0000x reference)
import jax
import jax.numpy as jnp
from jax.experimental import pallas as pl


def kernel(x, edge_index, W1, att_src1, att_dst1, bias1, W2, att_src2, att_dst2, bias2):
    raise NotImplementedError("write your pallas kernel here")


# speedup vs baseline: 46.7029x; 46.7029x over previous
import functools

import jax
import jax.numpy as jnp
from jax import lax
from jax.experimental import pallas as pl
from jax.experimental.pallas import tpu as pltpu
from jax.experimental.pallas import tpu_sc as plsc

_N = 10000
_E = 320000
_D_IN = 128
_HDIM = 64
_HEADS = 8

_NC = 2
_NS = 16
_NW = _NC * _NS
_EW = _E // _NW
_B = 80
_NB = _EW // _B
_U = 8


def _prep1(x, W1, As1, Ad1):
    def body(x_ref, w_ref, as_ref, ad_ref, S_ref, D_ref, ms_ref, md_ref):
        h = jnp.dot(x_ref[...], w_ref[...], preferred_element_type=jnp.float32)
        asrc = jnp.dot(h, as_ref[...], preferred_element_type=jnp.float32)
        adst = jnp.dot(h, ad_ref[...], preferred_element_type=jnp.float32)
        n = h.shape[0]
        ones8 = (lax.broadcasted_iota(jnp.int32, (n, 16), 1) < 8
                 ).astype(jnp.float32)
        S_ref[:, 0:64] = h
        S_ref[:, 64:80] = ones8
        S_ref[:, 80:96] = asrc
        D_ref[...] = adst
        ms_ref[...] = jnp.max(asrc, axis=0, keepdims=True)
        md_ref[...] = jnp.max(adst, axis=0, keepdims=True)

    return pl.pallas_call(
        body,
        out_shape=[
            jax.ShapeDtypeStruct((_N, 96), jnp.float32),
            jax.ShapeDtypeStruct((_N, 16), jnp.float32),
            jax.ShapeDtypeStruct((1, 16), jnp.float32),
            jax.ShapeDtypeStruct((1, 16), jnp.float32),
        ],
    )(x, W1, As1, Ad1)


def _prep2(acc1, Rep, b1, W2P, W2PD, Crow):
    def body(acc_ref, rep_ref, b1_ref, w2p_ref, w2pd_ref, c_ref,
             S_ref, D_ref, ms_ref, md_ref):
        a = acc_ref[0] + acc_ref[1]
        num = a[:, 0:64]
        den = a[:, 64:72]
        denE = jnp.dot(den, rep_ref[...], preferred_element_type=jnp.float32)
        h1 = jnp.maximum(num / (denE + 1e-16) + b1_ref[...], 0.0)
        S2 = jnp.dot(h1, w2p_ref[...], preferred_element_type=jnp.float32)
        S2 = S2 + c_ref[...]
        D2 = jnp.dot(h1, w2pd_ref[...], preferred_element_type=jnp.float32)
        S_ref[...] = S2
        D_ref[...] = D2
        ms_ref[...] = jnp.max(S2, axis=0, keepdims=True)
        md_ref[...] = jnp.max(D2, axis=0, keepdims=True)

    return pl.pallas_call(
        body,
        out_shape=[
            jax.ShapeDtypeStruct((_N, 16), jnp.float32),
            jax.ShapeDtypeStruct((_N, 16), jnp.float32),
            jax.ShapeDtypeStruct((1, 16), jnp.float32),
            jax.ShapeDtypeStruct((1, 16), jnp.float32),
        ],
    )(acc1, Rep, b1, W2P, W2PD, Crow)


def _finish(acc2, b2):
    def body(acc_ref, b2_ref, out_ref):
        a = acc_ref[0] + acc_ref[1]
        num = a[:, 0:2]
        den = a[:, 2:3]
        logit = num / (den + 1e-16) + b2_ref[...]
        m = jnp.max(logit, axis=1, keepdims=True)
        lse = m + jnp.log(jnp.sum(jnp.exp(logit - m), axis=1, keepdims=True))
        out_ref[...] = logit - lse

    return pl.pallas_call(
        body,
        out_shape=jax.ShapeDtypeStruct((_N, 2), jnp.float32),
    )(acc2, b2)


def _make_edge_pass(sw, aw, toff, idx_specs):
    nmsg = aw // 16
    zrows = 80
    nch = _N // zrows
    mesh = plsc.VectorSubcoreMesh(core_axis_name="c", subcore_axis_name="s",
                                  num_cores=_NC, num_subcores=_NS)

    @functools.partial(
        pl.kernel,
        out_type=jax.ShapeDtypeStruct((_NC, _N, aw), jnp.float32),
        mesh=mesh,
        compiler_params=pltpu.CompilerParams(needs_layout_passes=False,
                                             use_tc_tiling_on_sc=False),
        scratch_types=[
            pltpu.VMEM((_NB, _B), jnp.int32),
            pltpu.VMEM((_NB, _B), jnp.int32),
            pltpu.VMEM((_B, sw), jnp.float32),
            pltpu.VMEM((_B, 16), jnp.float32),
            pltpu.VMEM((_B, aw), jnp.float32),
            pltpu.VMEM((16,), jnp.float32),
            pltpu.VMEM((16,), jnp.float32),
            pltpu.VMEM((16,), jnp.float32),
            pltpu.VMEM((zrows, aw), jnp.float32),
            pltpu.VMEM_SHARED((_N, aw), jnp.float32),
        ],
    )
    def body(S_hbm, D_hbm, ms_hbm, md_hbm, src_hbm, dst_hbm, out_hbm,
             src_v, dst_v, s_buf, d_buf, m_buf, wbuf,
             msv, mdv, zbuf, acc):
        cid = lax.axis_index("c")
        sid = lax.axis_index("s")
        w_id = cid * _NS + sid

        def zb(i, carry):
            for c in range(nmsg):
                zbuf[i, pl.ds(16 * c, 16)] = jnp.zeros((16,), jnp.float32)
            return carry
        lax.fori_loop(0, zrows, zb, 0)
        for k in range((nch + _NS - 1) // _NS):
            ch = sid + k * _NS

            @pl.when(ch < nch)
            def _():
                pltpu.sync_copy(zbuf, acc.at[pl.ds(ch * zrows, zrows)])
        plsc.subcore_barrier()

        pltpu.sync_copy(ms_hbm, msv)
        pltpu.sync_copy(md_hbm, mdv)
        t = msv[...] + mdv[...]
        Mv = jnp.maximum(t, 0.2 * t)

        lanes = lax.iota(jnp.int32, 16)
        idxs = []
        for kind, val in idx_specs:
            if kind == "head":
                idxs.append((val * 16 + lanes) >> 3)
            elif kind == "id":
                idxs.append(lanes)
            else:
                idxs.append(lanes * 0 + val)

        pltpu.sync_copy(src_hbm.at[w_id], src_v)
        pltpu.sync_copy(dst_hbm.at[w_id], dst_v)

        def batch(j, carry):
            pltpu.sync_copy(S_hbm.at[src_v.at[j]], s_buf)
            pltpu.sync_copy(D_hbm.at[dst_v.at[j]], d_buf)

            def inner(jj, c2):
                for u in range(_U):
                    e_i = jj * _U + u
                    t = (s_buf[e_i, pl.ds(toff, 16)]
                         + d_buf[e_i, pl.ds(0, 16)])
                    t = jnp.maximum(t, 0.2 * t) - Mv
                    wbuf[...] = jnp.exp(t)
                    for k in range(nmsg):
                        wb = plsc.load_gather(wbuf, [idxs[k]])
                        m_buf[e_i, pl.ds(16 * k, 16)] = (
                            wb * s_buf[e_i, pl.ds(16 * k, 16)])
                return c2
            lax.fori_loop(0, _B // _U, inner, 0)

            pltpu.sync_copy(m_buf, acc.at[dst_v.at[j]], add=True)
            return carry
        lax.fori_loop(0, _NB, batch, 0)

        plsc.subcore_barrier()
        for k in range((nch + _NS - 1) // _NS):
            ch = sid + k * _NS

            @pl.when(ch < nch)
            def _():
                pltpu.sync_copy(acc.at[pl.ds(ch * zrows, zrows)], zbuf)
                pltpu.sync_copy(
                    zbuf, out_hbm.at[cid].at[pl.ds(ch * zrows, zrows)])

    return body


_SPECS1 = (("head", 0), ("head", 1), ("head", 2), ("head", 3), ("id", 0))
_SPECS2 = (("const", 8),)


@functools.lru_cache(maxsize=None)
def _get_edge_pass(sw, aw, toff, idx_specs):
    return _make_edge_pass(sw, aw, toff, idx_specs)


def kernel(x, edge_index, W1, att_src1, att_dst1, bias1,
           W2, att_src2, att_dst2, bias2):
    ei = edge_index.astype(jnp.int32)
    src = ei[0].reshape(_NW, _NB, _B)
    dst = ei[1].reshape(_NW, _NB, _B)

    f32 = jnp.float32
    heads = _HEADS
    oc = _HDIM // heads
    c64 = jnp.arange(_HDIM)
    j16 = jnp.arange(16)
    sel = (c64[:, None] // oc == j16[None, :]).astype(f32)
    As1 = sel * att_src1[0].reshape(-1)[:, None]
    Ad1 = sel * att_dst1[0].reshape(-1)[:, None]

    Rep = (jnp.arange(heads)[:, None] == (c64[None, :] // oc)).astype(f32)

    P = jnp.zeros((2, 16), f32)
    P = P.at[0, 0].set(1.0).at[1, 1].set(1.0)
    P = P.at[0, 8].set(att_src2[0, 0, 0]).at[1, 8].set(att_src2[0, 0, 1])
    PD = jnp.zeros((2, 16), f32)
    PD = PD.at[0, 8].set(att_dst2[0, 0, 0]).at[1, 8].set(att_dst2[0, 0, 1])
    W2P = W2 @ P
    W2PD = W2 @ PD
    Crow = jnp.zeros((1, 16), f32).at[0, 2].set(1.0)

    b1 = bias1.reshape(1, _HDIM)
    b2 = bias2.reshape(1, 2)

    S1, D1, ms1, md1 = _prep1(x, W1, As1, Ad1)
    acc1 = _get_edge_pass(96, 80, 80, _SPECS1)(
        S1, D1, ms1.reshape(16), md1.reshape(16), src, dst)
    S2, D2, ms2, md2 = _prep2(acc1, Rep, b1, W2P, W2PD, Crow)
    acc2 = _get_edge_pass(16, 16, 0, _SPECS2)(
        S2, D2, ms2.reshape(16), md2.reshape(16), src, dst)
    return _finish(acc2, b2)


# speedup vs baseline: 74.3754x; 1.5925x over previous
import functools

import jax
import jax.numpy as jnp
from jax import lax
from jax.experimental import pallas as pl
from jax.experimental.pallas import tpu as pltpu
from jax.experimental.pallas import tpu_sc as plsc

_N = 10000
_E = 320000
_D_IN = 128
_HDIM = 64
_HEADS = 8

_NC = 2
_NS = 16
_NW = _NC * _NS
_EW = _E // _NW
_B = 80
_NB = _EW // _B
_U = 8


def _prep1(x, W1, As1, Ad1):
    def body(x_ref, w_ref, as_ref, ad_ref, S_ref, D_ref, ms_ref, md_ref):
        h = jnp.dot(x_ref[...], w_ref[...], preferred_element_type=jnp.float32)
        asrc = jnp.dot(h, as_ref[...], preferred_element_type=jnp.float32)
        adst = jnp.dot(h, ad_ref[...], preferred_element_type=jnp.float32)
        n = h.shape[0]
        ones8 = (lax.broadcasted_iota(jnp.int32, (n, 16), 1) < 8
                 ).astype(jnp.float32)
        S_ref[:, 0:64] = h
        S_ref[:, 64:80] = ones8
        S_ref[:, 80:96] = asrc
        D_ref[...] = adst
        ms_ref[...] = jnp.max(asrc, axis=0, keepdims=True)
        md_ref[...] = jnp.max(adst, axis=0, keepdims=True)

    return pl.pallas_call(
        body,
        out_shape=[
            jax.ShapeDtypeStruct((_N, 96), jnp.float32),
            jax.ShapeDtypeStruct((_N, 16), jnp.float32),
            jax.ShapeDtypeStruct((1, 16), jnp.float32),
            jax.ShapeDtypeStruct((1, 16), jnp.float32),
        ],
    )(x, W1, As1, Ad1)


def _prep2(acc1, Rep, b1, W2P, W2PD, Crow):
    def body(acc_ref, rep_ref, b1_ref, w2p_ref, w2pd_ref, c_ref,
             S_ref, D_ref, ms_ref, md_ref):
        a = acc_ref[0] + acc_ref[1]
        num = a[:, 0:64]
        den = a[:, 64:72]
        denE = jnp.dot(den, rep_ref[...], preferred_element_type=jnp.float32)
        h1 = jnp.maximum(num / (denE + 1e-16) + b1_ref[...], 0.0)
        S2 = jnp.dot(h1, w2p_ref[...], preferred_element_type=jnp.float32)
        S2 = S2 + c_ref[...]
        D2 = jnp.dot(h1, w2pd_ref[...], preferred_element_type=jnp.float32)
        S_ref[...] = S2
        D_ref[...] = D2
        ms_ref[...] = jnp.max(S2, axis=0, keepdims=True)
        md_ref[...] = jnp.max(D2, axis=0, keepdims=True)

    return pl.pallas_call(
        body,
        out_shape=[
            jax.ShapeDtypeStruct((_N, 16), jnp.float32),
            jax.ShapeDtypeStruct((_N, 16), jnp.float32),
            jax.ShapeDtypeStruct((1, 16), jnp.float32),
            jax.ShapeDtypeStruct((1, 16), jnp.float32),
        ],
    )(acc1, Rep, b1, W2P, W2PD, Crow)


def _finish(acc2, b2):
    def body(acc_ref, b2_ref, out_ref):
        a = acc_ref[0] + acc_ref[1]
        num = a[:, 0:2]
        den = a[:, 2:3]
        logit = num / (den + 1e-16) + b2_ref[...]
        m = jnp.max(logit, axis=1, keepdims=True)
        lse = m + jnp.log(jnp.sum(jnp.exp(logit - m), axis=1, keepdims=True))
        out_ref[...] = logit - lse

    return pl.pallas_call(
        body,
        out_shape=jax.ShapeDtypeStruct((_N, 2), jnp.float32),
    )(acc2, b2)


def _make_edge_pass(sw, aw, toff, idx_specs, nbuf):
    nmsg = aw // 16
    zrows = 40
    nch = _N // zrows
    mesh = plsc.VectorSubcoreMesh(core_axis_name="c", subcore_axis_name="s",
                                  num_cores=_NC, num_subcores=_NS)

    @functools.partial(
        pl.kernel,
        out_type=jax.ShapeDtypeStruct((_NC, _N, aw), jnp.float32),
        mesh=mesh,
        compiler_params=pltpu.CompilerParams(needs_layout_passes=False,
                                             use_tc_tiling_on_sc=False),
        scratch_types=[
            pltpu.VMEM((_NB, _B), jnp.int32),
            pltpu.VMEM((_NB, _B), jnp.int32),
            pltpu.VMEM((16,), jnp.float32),
            pltpu.VMEM((16,), jnp.float32),
            pltpu.VMEM((16,), jnp.float32),
            pltpu.VMEM((zrows, aw), jnp.float32),
            pltpu.VMEM_SHARED((_N, aw), jnp.float32),
        ]
        + [pltpu.VMEM((_B, sw), jnp.float32) for _ in range(nbuf)]
        + [pltpu.VMEM((_B, 16), jnp.float32) for _ in range(nbuf)]
        + [pltpu.VMEM((_B, aw), jnp.float32) for _ in range(nbuf)]
        + [pltpu.SemaphoreType.DMA for _ in range(3 * nbuf)],
    )
    def body(S_hbm, D_hbm, ms_hbm, md_hbm, src_hbm, dst_hbm, out_hbm,
             src_v, dst_v, wbuf, msv, mdv, zbuf, acc, *bufs):
        s_bufs = bufs[0:nbuf]
        d_bufs = bufs[nbuf:2 * nbuf]
        m_bufs = bufs[2 * nbuf:3 * nbuf]
        sem_s = bufs[3 * nbuf:4 * nbuf]
        sem_d = bufs[4 * nbuf:5 * nbuf]
        sem_m = bufs[5 * nbuf:6 * nbuf]
        cid = lax.axis_index("c")
        sid = lax.axis_index("s")
        w_id = cid * _NS + sid

        def zb(i, carry):
            for c in range(nmsg):
                zbuf[i, pl.ds(16 * c, 16)] = jnp.zeros((16,), jnp.float32)
            return carry
        lax.fori_loop(0, zrows, zb, 0)
        for k in range((nch + _NS - 1) // _NS):
            ch = sid + k * _NS

            @pl.when(ch < nch)
            def _():
                pltpu.sync_copy(zbuf, acc.at[pl.ds(ch * zrows, zrows)])
        plsc.subcore_barrier()

        pltpu.sync_copy(ms_hbm, msv)
        pltpu.sync_copy(md_hbm, mdv)
        t = msv[...] + mdv[...]
        Mv = jnp.maximum(t, 0.2 * t)

        lanes = lax.iota(jnp.int32, 16)
        idxs = []
        for kind, val in idx_specs:
            if kind == "head":
                idxs.append((val * 16 + lanes) >> 3)
            elif kind == "id":
                idxs.append(lanes)
            else:
                idxs.append(lanes * 0 + val)

        pltpu.sync_copy(src_hbm.at[w_id], src_v)
        pltpu.sync_copy(dst_hbm.at[w_id], dst_v)

        def gstart(j, b):
            pltpu.async_copy(S_hbm.at[src_v.at[j]], s_bufs[b], sem_s[b])
            pltpu.async_copy(D_hbm.at[dst_v.at[j]], d_bufs[b], sem_d[b])

        def gwait(j, b):
            pltpu.make_async_copy(
                S_hbm.at[src_v.at[j]], s_bufs[b], sem_s[b]).wait()
            pltpu.make_async_copy(
                D_hbm.at[dst_v.at[j]], d_bufs[b], sem_d[b]).wait()

        def sstart(j, b):
            pltpu.async_copy(m_bufs[b], acc.at[dst_v.at[j]], sem_m[b],
                             add=True)

        def swait(b):
            pltpu.make_async_copy(
                m_bufs[b], acc.at[dst_v.at[0]], sem_m[b]).wait()

        def compute(b):
            s_buf = s_bufs[b]
            d_buf = d_bufs[b]
            m_buf = m_bufs[b]

            def inner(jj, c2):
                for u in range(_U):
                    e_i = jj * _U + u
                    t = (s_buf[e_i, pl.ds(toff, 16)]
                         + d_buf[e_i, pl.ds(0, 16)])
                    t = jnp.maximum(t, 0.2 * t) - Mv
                    wbuf[...] = jnp.exp(t)
                    for k in range(nmsg):
                        wb = plsc.load_gather(wbuf, [idxs[k]])
                        m_buf[e_i, pl.ds(16 * k, 16)] = (
                            wb * s_buf[e_i, pl.ds(16 * k, 16)])
                return c2
            lax.fori_loop(0, _B // _U, inner, 0)

        for b in range(nbuf):
            gstart(b, b)

        main_iters = _NB // nbuf

        def outer(j0, carry):
            for b in range(nbuf):
                j = j0 * nbuf + b
                gwait(j, b)

                @pl.when(j >= nbuf)
                def _():
                    swait(b)
                compute(b)
                sstart(j, b)

                @pl.when(j + nbuf < _NB)
                def _():
                    gstart(j + nbuf, b)
            return carry
        lax.fori_loop(0, main_iters, outer, 0)

        for j in range(main_iters * nbuf, _NB):
            b = j % nbuf
            gwait(j, b)
            swait(b)
            compute(b)
            sstart(j, b)
        for b in range(nbuf):
            swait(b)

        plsc.subcore_barrier()
        for k in range((nch + _NS - 1) // _NS):
            ch = sid + k * _NS

            @pl.when(ch < nch)
            def _():
                pltpu.sync_copy(acc.at[pl.ds(ch * zrows, zrows)], zbuf)
                pltpu.sync_copy(
                    zbuf, out_hbm.at[cid].at[pl.ds(ch * zrows, zrows)])

    return body


_SPECS1 = (("head", 0), ("head", 1), ("head", 2), ("head", 3), ("id", 0))
_SPECS2 = (("const", 8),)


@functools.lru_cache(maxsize=None)
def _get_edge_pass(sw, aw, toff, idx_specs, nbuf):
    return _make_edge_pass(sw, aw, toff, idx_specs, nbuf)


def kernel(x, edge_index, W1, att_src1, att_dst1, bias1,
           W2, att_src2, att_dst2, bias2):
    ei = edge_index.astype(jnp.int32)
    src = ei[0].reshape(_NW, _NB, _B)
    dst = ei[1].reshape(_NW, _NB, _B)

    f32 = jnp.float32
    heads = _HEADS
    oc = _HDIM // heads
    c64 = jnp.arange(_HDIM)
    j16 = jnp.arange(16)
    sel = (c64[:, None] // oc == j16[None, :]).astype(f32)
    As1 = sel * att_src1[0].reshape(-1)[:, None]
    Ad1 = sel * att_dst1[0].reshape(-1)[:, None]

    Rep = (jnp.arange(heads)[:, None] == (c64[None, :] // oc)).astype(f32)

    P = jnp.zeros((2, 16), f32)
    P = P.at[0, 0].set(1.0).at[1, 1].set(1.0)
    P = P.at[0, 8].set(att_src2[0, 0, 0]).at[1, 8].set(att_src2[0, 0, 1])
    PD = jnp.zeros((2, 16), f32)
    PD = PD.at[0, 8].set(att_dst2[0, 0, 0]).at[1, 8].set(att_dst2[0, 0, 1])
    W2P = W2 @ P
    W2PD = W2 @ PD
    Crow = jnp.zeros((1, 16), f32).at[0, 2].set(1.0)

    b1 = bias1.reshape(1, _HDIM)
    b2 = bias2.reshape(1, 2)

    S1, D1, ms1, md1 = _prep1(x, W1, As1, Ad1)
    acc1 = _get_edge_pass(96, 80, 80, _SPECS1, 3)(
        S1, D1, ms1.reshape(16), md1.reshape(16), src, dst)
    S2, D2, ms2, md2 = _prep2(acc1, Rep, b1, W2P, W2PD, Crow)
    acc2 = _get_edge_pass(16, 16, 0, _SPECS2, 4)(
        S2, D2, ms2.reshape(16), md2.reshape(16), src, dst)
    return _finish(acc2, b2)


# speedup vs baseline: 80.3818x; 1.0808x over previous
import functools

import jax
import jax.numpy as jnp
from jax import lax
from jax.experimental import pallas as pl
from jax.experimental.pallas import tpu as pltpu
from jax.experimental.pallas import tpu_sc as plsc

_N = 10000
_E = 320000
_D_IN = 128
_HDIM = 64
_HEADS = 8

_NC = 2
_NS = 16
_NW = _NC * _NS
_EW = _E // _NW
_B = 80
_NB = _EW // _B
_U = 8


def _prep1(x, W1, As1, Ad1):
    def body(x_ref, w_ref, as_ref, ad_ref, S_ref, D_ref, ms_ref, md_ref):
        h = jnp.dot(x_ref[...], w_ref[...], preferred_element_type=jnp.float32)
        asrc = jnp.dot(h, as_ref[...], preferred_element_type=jnp.float32)
        adst = jnp.dot(h, ad_ref[...], preferred_element_type=jnp.float32)
        n = h.shape[0]
        ones8 = (lax.broadcasted_iota(jnp.int32, (n, 16), 1) < 8
                 ).astype(jnp.float32)
        S_ref[:, 0:64] = h
        S_ref[:, 64:80] = ones8
        S_ref[:, 80:96] = asrc
        D_ref[...] = adst
        ms_ref[...] = jnp.max(asrc, axis=0, keepdims=True)
        md_ref[...] = jnp.max(adst, axis=0, keepdims=True)

    return pl.pallas_call(
        body,
        out_shape=[
            jax.ShapeDtypeStruct((_N, 96), jnp.float32),
            jax.ShapeDtypeStruct((_N, 16), jnp.float32),
            jax.ShapeDtypeStruct((1, 16), jnp.float32),
            jax.ShapeDtypeStruct((1, 16), jnp.float32),
        ],
    )(x, W1, As1, Ad1)


def _prep2(acc1, Rep, b1, W2P, W2PD, Crow):
    def body(acc_ref, rep_ref, b1_ref, w2p_ref, w2pd_ref, c_ref,
             S_ref, D_ref, ms_ref, md_ref):
        a = acc_ref[0] + acc_ref[1]
        num = a[:, 0:64]
        den = a[:, 64:72]
        denE = jnp.dot(den, rep_ref[...], preferred_element_type=jnp.float32)
        h1 = jnp.maximum(num / (denE + 1e-16) + b1_ref[...], 0.0)
        S2 = jnp.dot(h1, w2p_ref[...], preferred_element_type=jnp.float32)
        S2 = S2 + c_ref[...]
        D2 = jnp.dot(h1, w2pd_ref[...], preferred_element_type=jnp.float32)
        S_ref[...] = S2
        D_ref[...] = D2
        ms_ref[...] = jnp.max(S2, axis=0, keepdims=True)
        md_ref[...] = jnp.max(D2, axis=0, keepdims=True)

    return pl.pallas_call(
        body,
        out_shape=[
            jax.ShapeDtypeStruct((_N, 16), jnp.float32),
            jax.ShapeDtypeStruct((_N, 16), jnp.float32),
            jax.ShapeDtypeStruct((1, 16), jnp.float32),
            jax.ShapeDtypeStruct((1, 16), jnp.float32),
        ],
    )(acc1, Rep, b1, W2P, W2PD, Crow)


def _finish(acc2, b2):
    def body(acc_ref, b2_ref, out_ref):
        a = acc_ref[0] + acc_ref[1]
        num = a[:, 0:2]
        den = a[:, 2:3]
        logit = num / (den + 1e-16) + b2_ref[...]
        m = jnp.max(logit, axis=1, keepdims=True)
        lse = m + jnp.log(jnp.sum(jnp.exp(logit - m), axis=1, keepdims=True))
        out_ref[...] = logit - lse

    return pl.pallas_call(
        body,
        out_shape=jax.ShapeDtypeStruct((_N, 2), jnp.float32),
    )(acc2, b2)


def _make_edge_pass(sw, aw, toff, idx_specs, nbuf):
    nmsg = aw // 16
    zrows = 40
    nch = _N // zrows
    mesh = plsc.VectorSubcoreMesh(core_axis_name="c", subcore_axis_name="s",
                                  num_cores=_NC, num_subcores=_NS)

    @functools.partial(
        pl.kernel,
        out_type=jax.ShapeDtypeStruct((_NC, _N, aw), jnp.float32),
        mesh=mesh,
        compiler_params=pltpu.CompilerParams(needs_layout_passes=False,
                                             use_tc_tiling_on_sc=False),
        scratch_types=[
            pltpu.VMEM((_NB, _B), jnp.int32),
            pltpu.VMEM((_NB, _B), jnp.int32),
            pltpu.VMEM((_U * 16,), jnp.float32),
            pltpu.VMEM((16,), jnp.float32),
            pltpu.VMEM((16,), jnp.float32),
            pltpu.VMEM((zrows, aw), jnp.float32),
            pltpu.VMEM_SHARED((_N, aw), jnp.float32),
        ]
        + [pltpu.VMEM((_B, sw), jnp.float32) for _ in range(nbuf)]
        + [pltpu.VMEM((_B, 16), jnp.float32) for _ in range(nbuf)]
        + [pltpu.VMEM((_B, aw), jnp.float32) for _ in range(nbuf)]
        + [pltpu.SemaphoreType.DMA for _ in range(3 * nbuf)],
    )
    def body(S_hbm, D_hbm, ms_hbm, md_hbm, src_hbm, dst_hbm, out_hbm,
             src_v, dst_v, wbuf, msv, mdv, zbuf, acc, *bufs):
        s_bufs = bufs[0:nbuf]
        d_bufs = bufs[nbuf:2 * nbuf]
        m_bufs = bufs[2 * nbuf:3 * nbuf]
        sem_s = bufs[3 * nbuf:4 * nbuf]
        sem_d = bufs[4 * nbuf:5 * nbuf]
        sem_m = bufs[5 * nbuf:6 * nbuf]
        cid = lax.axis_index("c")
        sid = lax.axis_index("s")
        w_id = cid * _NS + sid

        def zb(i, carry):
            for c in range(nmsg):
                zbuf[i, pl.ds(16 * c, 16)] = jnp.zeros((16,), jnp.float32)
            return carry
        lax.fori_loop(0, zrows, zb, 0)
        for k in range((nch + _NS - 1) // _NS):
            ch = sid + k * _NS

            @pl.when(ch < nch)
            def _():
                pltpu.sync_copy(zbuf, acc.at[pl.ds(ch * zrows, zrows)])
        plsc.subcore_barrier()

        pltpu.sync_copy(ms_hbm, msv)
        pltpu.sync_copy(md_hbm, mdv)
        t = msv[...] + mdv[...]
        Mv = jnp.maximum(t, 0.2 * t)

        lanes = lax.iota(jnp.int32, 16)
        idxs = []
        for kind, val in idx_specs:
            if kind == "head":
                idxs.append((val * 16 + lanes) >> 3)
            elif kind == "id":
                idxs.append(None)
            else:
                idxs.append(lanes * 0 + val)
        maskv = (lanes < 8).astype(jnp.float32)

        pltpu.sync_copy(src_hbm.at[w_id], src_v)
        pltpu.sync_copy(dst_hbm.at[w_id], dst_v)

        def gstart(j, b):
            pltpu.async_copy(S_hbm.at[src_v.at[j]], s_bufs[b], sem_s[b])
            pltpu.async_copy(D_hbm.at[dst_v.at[j]], d_bufs[b], sem_d[b])

        def gwait(j, b):
            pltpu.make_async_copy(
                S_hbm.at[src_v.at[j]], s_bufs[b], sem_s[b]).wait()
            pltpu.make_async_copy(
                D_hbm.at[dst_v.at[j]], d_bufs[b], sem_d[b]).wait()

        def sstart(j, b):
            pltpu.async_copy(m_bufs[b], acc.at[dst_v.at[j]], sem_m[b],
                             add=True)

        def swait(b):
            pltpu.make_async_copy(
                m_bufs[b], acc.at[dst_v.at[0]], sem_m[b]).wait()

        def compute(b):
            s_buf = s_bufs[b]
            d_buf = d_bufs[b]
            m_buf = m_bufs[b]

            def inner(jj, c2):
                for u in range(_U):
                    e_i = jj * _U + u
                    off = 16 * u
                    t = (s_buf[e_i, pl.ds(toff, 16)]
                         + d_buf[e_i, pl.ds(0, 16)])
                    t = jnp.maximum(t, 0.2 * t) - Mv
                    wv = jnp.exp(t)
                    if any(ix is not None for ix in idxs):
                        wbuf[pl.ds(off, 16)] = wv
                    for k in range(nmsg):
                        if idxs[k] is None:
                            m = wv * maskv
                        else:
                            wb = plsc.load_gather(wbuf, [idxs[k] + off])
                            m = wb * s_buf[e_i, pl.ds(16 * k, 16)]
                        m_buf[e_i, pl.ds(16 * k, 16)] = m
                return c2
            lax.fori_loop(0, _B // _U, inner, 0)

        for b in range(nbuf):
            gstart(b, b)

        main_iters = _NB // nbuf

        def outer(j0, carry):
            for b in range(nbuf):
                j = j0 * nbuf + b
                gwait(j, b)

                @pl.when(j >= nbuf)
                def _():
                    swait(b)
                compute(b)
                sstart(j, b)

                @pl.when(j + nbuf < _NB)
                def _():
                    gstart(j + nbuf, b)
            return carry
        lax.fori_loop(0, main_iters, outer, 0)

        for j in range(main_iters * nbuf, _NB):
            b = j % nbuf
            gwait(j, b)
            swait(b)
            compute(b)
            sstart(j, b)
        for b in range(nbuf):
            swait(b)

        plsc.subcore_barrier()
        for k in range((nch + _NS - 1) // _NS):
            ch = sid + k * _NS

            @pl.when(ch < nch)
            def _():
                pltpu.sync_copy(acc.at[pl.ds(ch * zrows, zrows)], zbuf)
                pltpu.sync_copy(
                    zbuf, out_hbm.at[cid].at[pl.ds(ch * zrows, zrows)])

    return body


_SPECS1 = (("head", 0), ("head", 1), ("head", 2), ("head", 3), ("id", 0))
_SPECS2 = (("const", 8),)


@functools.lru_cache(maxsize=None)
def _get_edge_pass(sw, aw, toff, idx_specs, nbuf):
    return _make_edge_pass(sw, aw, toff, idx_specs, nbuf)


def kernel(x, edge_index, W1, att_src1, att_dst1, bias1,
           W2, att_src2, att_dst2, bias2):
    ei = edge_index.astype(jnp.int32)
    src = ei[0].reshape(_NW, _NB, _B)
    dst = ei[1].reshape(_NW, _NB, _B)

    f32 = jnp.float32
    heads = _HEADS
    oc = _HDIM // heads
    c64 = jnp.arange(_HDIM)
    j16 = jnp.arange(16)
    sel = (c64[:, None] // oc == j16[None, :]).astype(f32)
    As1 = sel * att_src1[0].reshape(-1)[:, None]
    Ad1 = sel * att_dst1[0].reshape(-1)[:, None]

    Rep = (jnp.arange(heads)[:, None] == (c64[None, :] // oc)).astype(f32)

    P = jnp.zeros((2, 16), f32)
    P = P.at[0, 0].set(1.0).at[1, 1].set(1.0)
    P = P.at[0, 8].set(att_src2[0, 0, 0]).at[1, 8].set(att_src2[0, 0, 1])
    PD = jnp.zeros((2, 16), f32)
    PD = PD.at[0, 8].set(att_dst2[0, 0, 0]).at[1, 8].set(att_dst2[0, 0, 1])
    W2P = W2 @ P
    W2PD = W2 @ PD
    Crow = jnp.zeros((1, 16), f32).at[0, 2].set(1.0)

    b1 = bias1.reshape(1, _HDIM)
    b2 = bias2.reshape(1, 2)

    S1, D1, ms1, md1 = _prep1(x, W1, As1, Ad1)
    acc1 = _get_edge_pass(96, 80, 80, _SPECS1, 3)(
        S1, D1, ms1.reshape(16), md1.reshape(16), src, dst)
    S2, D2, ms2, md2 = _prep2(acc1, Rep, b1, W2P, W2PD, Crow)
    acc2 = _get_edge_pass(16, 16, 0, _SPECS2, 4)(
        S2, D2, ms2.reshape(16), md2.reshape(16), src, dst)
    return _finish(acc2, b2)


# speedup vs baseline: 82.8466x; 1.0307x over previous
import functools

import jax
import jax.numpy as jnp
from jax import lax
from jax.experimental import pallas as pl
from jax.experimental.pallas import tpu as pltpu
from jax.experimental.pallas import tpu_sc as plsc

_N = 10000
_E = 320000
_D_IN = 128
_HDIM = 64
_HEADS = 8

_NC = 2
_NS = 16
_NW = _NC * _NS
_EW = _E // _NW
_B = 100
_NB = _EW // _B
_U = 10


def _prep1(x, W1, As1, Ad1):
    def body(x_ref, w_ref, as_ref, ad_ref, S_ref, D_ref, ms_ref, md_ref):
        h = jnp.dot(x_ref[...], w_ref[...], preferred_element_type=jnp.float32)
        asrc = jnp.dot(h, as_ref[...], preferred_element_type=jnp.float32)
        adst = jnp.dot(h, ad_ref[...], preferred_element_type=jnp.float32)
        S_ref[:, 0:64] = h
        S_ref[:, 64:80] = asrc
        D_ref[...] = adst
        ms_ref[...] = jnp.max(asrc, axis=0, keepdims=True)
        md_ref[...] = jnp.max(adst, axis=0, keepdims=True)

    return pl.pallas_call(
        body,
        out_shape=[
            jax.ShapeDtypeStruct((_N, 80), jnp.float32),
            jax.ShapeDtypeStruct((_N, 16), jnp.float32),
            jax.ShapeDtypeStruct((1, 16), jnp.float32),
            jax.ShapeDtypeStruct((1, 16), jnp.float32),
        ],
    )(x, W1, As1, Ad1)


def _prep2(acc1, Rep, b1, W2P, W2PD, Crow):
    def body(acc_ref, rep_ref, b1_ref, w2p_ref, w2pd_ref, c_ref,
             S_ref, D_ref, ms_ref, md_ref):
        a = acc_ref[0] + acc_ref[1]
        num = a[:, 0:64]
        den = a[:, 64:72]
        denE = jnp.dot(den, rep_ref[...], preferred_element_type=jnp.float32)
        h1 = jnp.maximum(num / (denE + 1e-16) + b1_ref[...], 0.0)
        S2 = jnp.dot(h1, w2p_ref[...], preferred_element_type=jnp.float32)
        S2 = S2 + c_ref[...]
        D2 = jnp.dot(h1, w2pd_ref[...], preferred_element_type=jnp.float32)
        S_ref[...] = S2
        D_ref[...] = D2
        ms_ref[...] = jnp.max(S2, axis=0, keepdims=True)
        md_ref[...] = jnp.max(D2, axis=0, keepdims=True)

    return pl.pallas_call(
        body,
        out_shape=[
            jax.ShapeDtypeStruct((_N, 16), jnp.float32),
            jax.ShapeDtypeStruct((_N, 16), jnp.float32),
            jax.ShapeDtypeStruct((1, 16), jnp.float32),
            jax.ShapeDtypeStruct((1, 16), jnp.float32),
        ],
    )(acc1, Rep, b1, W2P, W2PD, Crow)


def _finish(acc2, b2):
    def body(acc_ref, b2_ref, out_ref):
        a = acc_ref[0] + acc_ref[1]
        num = a[:, 0:2]
        den = a[:, 2:3]
        logit = num / (den + 1e-16) + b2_ref[...]
        m = jnp.max(logit, axis=1, keepdims=True)
        lse = m + jnp.log(jnp.sum(jnp.exp(logit - m), axis=1, keepdims=True))
        out_ref[...] = logit - lse

    return pl.pallas_call(
        body,
        out_shape=jax.ShapeDtypeStruct((_N, 2), jnp.float32),
    )(acc2, b2)


def _make_edge_pass(sw, aw, toff, idx_specs, nbuf):
    nmsg = aw // 16
    zrows = 40
    nch = _N // zrows
    mesh = plsc.VectorSubcoreMesh(core_axis_name="c", subcore_axis_name="s",
                                  num_cores=_NC, num_subcores=_NS)

    @functools.partial(
        pl.kernel,
        out_type=jax.ShapeDtypeStruct((_NC, _N, aw), jnp.float32),
        mesh=mesh,
        compiler_params=pltpu.CompilerParams(needs_layout_passes=False,
                                             use_tc_tiling_on_sc=False),
        scratch_types=[
            pltpu.VMEM((_NB, _B), jnp.int32),
            pltpu.VMEM((_NB, _B), jnp.int32),
            pltpu.VMEM((_U * 16,), jnp.float32),
            pltpu.VMEM((16,), jnp.float32),
            pltpu.VMEM((16,), jnp.float32),
            pltpu.VMEM((zrows, aw), jnp.float32),
            pltpu.VMEM_SHARED((_N, aw), jnp.float32),
        ]
        + [pltpu.VMEM((_B, sw), jnp.float32) for _ in range(nbuf)]
        + [pltpu.VMEM((_B, 16), jnp.float32) for _ in range(nbuf)]
        + [pltpu.VMEM((_B, aw), jnp.float32) for _ in range(nbuf)]
        + [pltpu.SemaphoreType.DMA for _ in range(3 * nbuf)],
    )
    def body(S_hbm, D_hbm, ms_hbm, md_hbm, src_hbm, dst_hbm, out_hbm,
             src_v, dst_v, wbuf, msv, mdv, zbuf, acc, *bufs):
        s_bufs = bufs[0:nbuf]
        d_bufs = bufs[nbuf:2 * nbuf]
        m_bufs = bufs[2 * nbuf:3 * nbuf]
        sem_s = bufs[3 * nbuf:4 * nbuf]
        sem_d = bufs[4 * nbuf:5 * nbuf]
        sem_m = bufs[5 * nbuf:6 * nbuf]
        cid = lax.axis_index("c")
        sid = lax.axis_index("s")
        w_id = cid * _NS + sid

        def zb(i, carry):
            for c in range(nmsg):
                zbuf[i, pl.ds(16 * c, 16)] = jnp.zeros((16,), jnp.float32)
            return carry
        lax.fori_loop(0, zrows, zb, 0)
        for k in range((nch + _NS - 1) // _NS):
            ch = sid + k * _NS

            @pl.when(ch < nch)
            def _():
                pltpu.sync_copy(zbuf, acc.at[pl.ds(ch * zrows, zrows)])
        plsc.subcore_barrier()

        pltpu.sync_copy(ms_hbm, msv)
        pltpu.sync_copy(md_hbm, mdv)
        t = msv[...] + mdv[...]
        Mv = jnp.maximum(t, 0.2 * t)

        lanes = lax.iota(jnp.int32, 16)
        idxs = []
        for kind, val in idx_specs:
            if kind == "head":
                idxs.append((val * 16 + lanes) >> 3)
            elif kind == "id":
                idxs.append(None)
            else:
                idxs.append(lanes * 0 + val)
        maskv = (lanes < 8).astype(jnp.float32)

        pltpu.sync_copy(src_hbm.at[w_id], src_v)
        pltpu.sync_copy(dst_hbm.at[w_id], dst_v)

        def gstart(j, b):
            pltpu.async_copy(S_hbm.at[src_v.at[j]], s_bufs[b], sem_s[b])
            pltpu.async_copy(D_hbm.at[dst_v.at[j]], d_bufs[b], sem_d[b])

        def gwait(j, b):
            pltpu.make_async_copy(
                S_hbm.at[src_v.at[j]], s_bufs[b], sem_s[b]).wait()
            pltpu.make_async_copy(
                D_hbm.at[dst_v.at[j]], d_bufs[b], sem_d[b]).wait()

        def sstart(j, b):
            pltpu.async_copy(m_bufs[b], acc.at[dst_v.at[j]], sem_m[b],
                             add=True)

        def swait(b):
            pltpu.make_async_copy(
                m_bufs[b], acc.at[dst_v.at[0]], sem_m[b]).wait()

        def compute(b):
            s_buf = s_bufs[b]
            d_buf = d_bufs[b]
            m_buf = m_bufs[b]

            def inner(jj, c2):
                for u in range(_U):
                    e_i = jj * _U + u
                    off = 16 * u
                    t = (s_buf[e_i, pl.ds(toff, 16)]
                         + d_buf[e_i, pl.ds(0, 16)])
                    t = jnp.maximum(t, 0.2 * t) - Mv
                    wv = jnp.exp(t)
                    if any(ix is not None for ix in idxs):
                        wbuf[pl.ds(off, 16)] = wv
                    for k in range(nmsg):
                        if idxs[k] is None:
                            m = wv * maskv
                        else:
                            wb = plsc.load_gather(wbuf, [idxs[k] + off])
                            m = wb * s_buf[e_i, pl.ds(16 * k, 16)]
                        m_buf[e_i, pl.ds(16 * k, 16)] = m
                return c2
            lax.fori_loop(0, _B // _U, inner, 0)

        for b in range(nbuf):
            gstart(b, b)

        main_iters = _NB // nbuf

        def outer(j0, carry):
            for b in range(nbuf):
                j = j0 * nbuf + b
                gwait(j, b)

                @pl.when(j >= nbuf)
                def _():
                    swait(b)
                compute(b)
                sstart(j, b)

                @pl.when(j + nbuf < _NB)
                def _():
                    gstart(j + nbuf, b)
            return carry
        lax.fori_loop(0, main_iters, outer, 0)

        for j in range(main_iters * nbuf, _NB):
            b = j % nbuf
            gwait(j, b)
            swait(b)
            compute(b)
            sstart(j, b)
        for b in range(nbuf):
            swait(b)

        plsc.subcore_barrier()
        for k in range((nch + _NS - 1) // _NS):
            ch = sid + k * _NS

            @pl.when(ch < nch)
            def _():
                pltpu.sync_copy(acc.at[pl.ds(ch * zrows, zrows)], zbuf)
                pltpu.sync_copy(
                    zbuf, out_hbm.at[cid].at[pl.ds(ch * zrows, zrows)])

    return body


_SPECS1 = (("head", 0), ("head", 1), ("head", 2), ("head", 3), ("id", 0))
_SPECS2 = (("const", 8),)


@functools.lru_cache(maxsize=None)
def _get_edge_pass(sw, aw, toff, idx_specs, nbuf):
    return _make_edge_pass(sw, aw, toff, idx_specs, nbuf)


def kernel(x, edge_index, W1, att_src1, att_dst1, bias1,
           W2, att_src2, att_dst2, bias2):
    ei = edge_index.astype(jnp.int32)
    src = ei[0].reshape(_NW, _NB, _B)
    dst = ei[1].reshape(_NW, _NB, _B)

    f32 = jnp.float32
    heads = _HEADS
    oc = _HDIM // heads
    c64 = jnp.arange(_HDIM)
    j16 = jnp.arange(16)
    sel = (c64[:, None] // oc == j16[None, :]).astype(f32)
    As1 = sel * att_src1[0].reshape(-1)[:, None]
    Ad1 = sel * att_dst1[0].reshape(-1)[:, None]

    Rep = (jnp.arange(heads)[:, None] == (c64[None, :] // oc)).astype(f32)

    P = jnp.zeros((2, 16), f32)
    P = P.at[0, 0].set(1.0).at[1, 1].set(1.0)
    P = P.at[0, 8].set(att_src2[0, 0, 0]).at[1, 8].set(att_src2[0, 0, 1])
    PD = jnp.zeros((2, 16), f32)
    PD = PD.at[0, 8].set(att_dst2[0, 0, 0]).at[1, 8].set(att_dst2[0, 0, 1])
    W2P = W2 @ P
    W2PD = W2 @ PD
    Crow = jnp.zeros((1, 16), f32).at[0, 2].set(1.0)

    b1 = bias1.reshape(1, _HDIM)
    b2 = bias2.reshape(1, 2)

    S1, D1, ms1, md1 = _prep1(x, W1, As1, Ad1)
    acc1 = _get_edge_pass(80, 80, 64, _SPECS1, 3)(
        S1, D1, ms1.reshape(16), md1.reshape(16), src, dst)
    S2, D2, ms2, md2 = _prep2(acc1, Rep, b1, W2P, W2PD, Crow)
    acc2 = _get_edge_pass(16, 16, 0, _SPECS2, 4)(
        S2, D2, ms2.reshape(16), md2.reshape(16), src, dst)
    return _finish(acc2, b2)


# speedup vs baseline: 105.9865x; 1.2793x over previous
import functools

import jax
import jax.numpy as jnp
from jax import lax
from jax.experimental import pallas as pl
from jax.experimental.pallas import tpu as pltpu
from jax.experimental.pallas import tpu_sc as plsc

_N = 10000
_E = 320000
_D_IN = 128
_HDIM = 64
_HEADS = 8

_NC = 2
_NS = 16
_NW = _NC * _NS
_EW = _E // _NW
_B = 100
_NB = _EW // _B
_U = 10


def _prep1(x, W1, As1, Ad1):
    def body(x_ref, w_ref, as_ref, ad_ref, S_ref, D_ref, ms_ref, md_ref):
        h = jnp.dot(x_ref[...], w_ref[...], preferred_element_type=jnp.float32)
        asrc = jnp.dot(h, as_ref[...], preferred_element_type=jnp.float32)
        adst = jnp.dot(h, ad_ref[...], preferred_element_type=jnp.float32)
        S_ref[:, 0:64] = h
        S_ref[:, 64:80] = asrc
        D_ref[...] = adst
        ms_ref[...] = jnp.max(asrc, axis=0, keepdims=True)
        md_ref[...] = jnp.max(adst, axis=0, keepdims=True)

    return pl.pallas_call(
        body,
        out_shape=[
            jax.ShapeDtypeStruct((_N, 80), jnp.float32),
            jax.ShapeDtypeStruct((_N, 16), jnp.float32),
            jax.ShapeDtypeStruct((1, 16), jnp.float32),
            jax.ShapeDtypeStruct((1, 16), jnp.float32),
        ],
    )(x, W1, As1, Ad1)


def _prep2(acc1, Rep, b1, W2Q):
    def body(acc_ref, rep_ref, b1_ref, w2q_ref, P_ref, ms_ref, md_ref):
        a = acc_ref[0] + acc_ref[1]
        num = a[:, 0:64]
        den = a[:, 64:72]
        denE = jnp.dot(den, rep_ref[...], preferred_element_type=jnp.float32)
        h1 = jnp.maximum(num / (denE + 1e-16) + b1_ref[...], 0.0)
        P2 = jnp.dot(h1, w2q_ref[...], preferred_element_type=jnp.float32)
        P_ref[...] = P2
        one = jnp.ones((1, 16), jnp.float32)
        ms_ref[...] = jnp.max(P2[:, 2]) * one
        md_ref[...] = jnp.max(P2[:, 3]) * one

    return pl.pallas_call(
        body,
        out_shape=[
            jax.ShapeDtypeStruct((_N, 4), jnp.float32),
            jax.ShapeDtypeStruct((1, 16), jnp.float32),
            jax.ShapeDtypeStruct((1, 16), jnp.float32),
        ],
    )(acc1, Rep, b1, W2Q)


def _finish(acc2, b2):
    def body(acc_ref, b2_ref, out_ref):
        a = jnp.sum(acc_ref[...], axis=0)
        den = a[2:3, :] + 1e-16
        l0 = a[0:1, :] / den + b2_ref[0, 0]
        l1 = a[1:2, :] / den + b2_ref[0, 1]
        m = jnp.maximum(l0, l1)
        lse = m + jnp.log(jnp.exp(l0 - m) + jnp.exp(l1 - m))
        out_ref[0:1, :] = l0 - lse
        out_ref[1:2, :] = l1 - lse

    return pl.pallas_call(
        body,
        out_shape=jax.ShapeDtypeStruct((2, _N), jnp.float32),
    )(acc2, b2)


def _make_edge_pass2():
    mesh = plsc.VectorSubcoreMesh(core_axis_name="c", subcore_axis_name="s",
                                  num_cores=_NC, num_subcores=_NS)

    @functools.partial(
        pl.kernel,
        out_type=jax.ShapeDtypeStruct((_NW, 3, _N), jnp.float32),
        mesh=mesh,
        compiler_params=pltpu.CompilerParams(needs_layout_passes=False,
                                             use_tc_tiling_on_sc=False),
        scratch_types=[
            pltpu.VMEM((_N, 4), jnp.float32),
            pltpu.VMEM((_NB, _B), jnp.int32),
            pltpu.VMEM((_NB, _B), jnp.int32),
            pltpu.VMEM((16,), jnp.float32),
            pltpu.VMEM((16,), jnp.float32),
            pltpu.VMEM((_N,), jnp.float32),
            pltpu.VMEM((_N,), jnp.float32),
            pltpu.VMEM((_N,), jnp.float32),
        ],
    )
    def body(P_hbm, ms_hbm, md_hbm, src_hbm, dst_hbm, out_hbm,
             Pv, src_v, dst_v, msv, mdv, acc0, acc1, acc2):
        cid = lax.axis_index("c")
        sid = lax.axis_index("s")
        w_id = cid * _NS + sid

        def zb(i, carry):
            z = jnp.zeros((16,), jnp.float32)
            acc0[pl.ds(i * 16, 16)] = z
            acc1[pl.ds(i * 16, 16)] = z
            acc2[pl.ds(i * 16, 16)] = z
            return carry
        lax.fori_loop(0, _N // 16, zb, 0)

        pltpu.sync_copy(P_hbm, Pv)
        pltpu.sync_copy(src_hbm.at[w_id], src_v)
        pltpu.sync_copy(dst_hbm.at[w_id], dst_v)
        pltpu.sync_copy(ms_hbm, msv)
        pltpu.sync_copy(md_hbm, mdv)
        t = msv[...] + mdv[...]
        Mv = jnp.maximum(t, 0.2 * t)

        zeros16 = lax.iota(jnp.int32, 16) * 0
        c1 = zeros16 + 1
        c2 = zeros16 + 2
        c3 = zeros16 + 3

        def batch(j, carry):
            for c in range(_B // 16):
                src16 = src_v[j, pl.ds(16 * c, 16)]
                dst16 = dst_v[j, pl.ds(16 * c, 16)]
                h0 = plsc.load_gather(Pv, [src16, zeros16])
                h1 = plsc.load_gather(Pv, [src16, c1])
                sa = plsc.load_gather(Pv, [src16, c2])
                da = plsc.load_gather(Pv, [dst16, c3])
                t = sa + da
                t = jnp.maximum(t, 0.2 * t) - Mv
                w = jnp.exp(t)
                plsc.addupdate_scatter(acc0, [dst16], w * h0)
                plsc.addupdate_scatter(acc1, [dst16], w * h1)
                plsc.addupdate_scatter(acc2, [dst16], w)
            return carry
        lax.fori_loop(0, _NB, batch, 0)

        pltpu.sync_copy(acc0, out_hbm.at[w_id].at[0])
        pltpu.sync_copy(acc1, out_hbm.at[w_id].at[1])
        pltpu.sync_copy(acc2, out_hbm.at[w_id].at[2])

    return body


def _make_edge_pass(sw, aw, toff, idx_specs, nbuf):
    nmsg = aw // 16
    zrows = 40
    nch = _N // zrows
    mesh = plsc.VectorSubcoreMesh(core_axis_name="c", subcore_axis_name="s",
                                  num_cores=_NC, num_subcores=_NS)

    @functools.partial(
        pl.kernel,
        out_type=jax.ShapeDtypeStruct((_NC, _N, aw), jnp.float32),
        mesh=mesh,
        compiler_params=pltpu.CompilerParams(needs_layout_passes=False,
                                             use_tc_tiling_on_sc=False),
        scratch_types=[
            pltpu.VMEM((_NB, _B), jnp.int32),
            pltpu.VMEM((_NB, _B), jnp.int32),
            pltpu.VMEM((_U * 16,), jnp.float32),
            pltpu.VMEM((16,), jnp.float32),
            pltpu.VMEM((16,), jnp.float32),
            pltpu.VMEM((zrows, aw), jnp.float32),
            pltpu.VMEM_SHARED((_N, aw), jnp.float32),
        ]
        + [pltpu.VMEM((_B, sw), jnp.float32) for _ in range(nbuf)]
        + [pltpu.VMEM((_B, 16), jnp.float32) for _ in range(nbuf)]
        + [pltpu.VMEM((_B, aw), jnp.float32) for _ in range(nbuf)]
        + [pltpu.SemaphoreType.DMA for _ in range(3 * nbuf)],
    )
    def body(S_hbm, D_hbm, ms_hbm, md_hbm, src_hbm, dst_hbm, out_hbm,
             src_v, dst_v, wbuf, msv, mdv, zbuf, acc, *bufs):
        s_bufs = bufs[0:nbuf]
        d_bufs = bufs[nbuf:2 * nbuf]
        m_bufs = bufs[2 * nbuf:3 * nbuf]
        sem_s = bufs[3 * nbuf:4 * nbuf]
        sem_d = bufs[4 * nbuf:5 * nbuf]
        sem_m = bufs[5 * nbuf:6 * nbuf]
        cid = lax.axis_index("c")
        sid = lax.axis_index("s")
        w_id = cid * _NS + sid

        def zb(i, carry):
            for c in range(nmsg):
                zbuf[i, pl.ds(16 * c, 16)] = jnp.zeros((16,), jnp.float32)
            return carry
        lax.fori_loop(0, zrows, zb, 0)
        for k in range((nch + _NS - 1) // _NS):
            ch = sid + k * _NS

            @pl.when(ch < nch)
            def _():
                pltpu.sync_copy(zbuf, acc.at[pl.ds(ch * zrows, zrows)])
        plsc.subcore_barrier()

        pltpu.sync_copy(ms_hbm, msv)
        pltpu.sync_copy(md_hbm, mdv)
        t = msv[...] + mdv[...]
        Mv = jnp.maximum(t, 0.2 * t)

        lanes = lax.iota(jnp.int32, 16)
        idxs = []
        for kind, val in idx_specs:
            if kind == "head":
                idxs.append((val * 16 + lanes) >> 3)
            elif kind == "id":
                idxs.append(None)
            else:
                idxs.append(lanes * 0 + val)
        maskv = (lanes < 8).astype(jnp.float32)

        pltpu.sync_copy(src_hbm.at[w_id], src_v)
        pltpu.sync_copy(dst_hbm.at[w_id], dst_v)

        def gstart(j, b):
            pltpu.async_copy(S_hbm.at[src_v.at[j]], s_bufs[b], sem_s[b])
            pltpu.async_copy(D_hbm.at[dst_v.at[j]], d_bufs[b], sem_d[b])

        def gwait(j, b):
            pltpu.make_async_copy(
                S_hbm.at[src_v.at[j]], s_bufs[b], sem_s[b]).wait()
            pltpu.make_async_copy(
                D_hbm.at[dst_v.at[j]], d_bufs[b], sem_d[b]).wait()

        def sstart(j, b):
            pltpu.async_copy(m_bufs[b], acc.at[dst_v.at[j]], sem_m[b],
                             add=True)

        def swait(b):
            pltpu.make_async_copy(
                m_bufs[b], acc.at[dst_v.at[0]], sem_m[b]).wait()

        def compute(b):
            s_buf = s_bufs[b]
            d_buf = d_bufs[b]
            m_buf = m_bufs[b]

            def inner(jj, c2):
                for u in range(_U):
                    e_i = jj * _U + u
                    off = 16 * u
                    t = (s_buf[e_i, pl.ds(toff, 16)]
                         + d_buf[e_i, pl.ds(0, 16)])
                    t = jnp.maximum(t, 0.2 * t) - Mv
                    wv = jnp.exp(t)
                    if any(ix is not None for ix in idxs):
                        wbuf[pl.ds(off, 16)] = wv
                    for k in range(nmsg):
                        if idxs[k] is None:
                            m = wv * maskv
                        else:
                            wb = plsc.load_gather(wbuf, [idxs[k] + off])
                            m = wb * s_buf[e_i, pl.ds(16 * k, 16)]
                        m_buf[e_i, pl.ds(16 * k, 16)] = m
                return c2
            lax.fori_loop(0, _B // _U, inner, 0)

        for b in range(nbuf):
            gstart(b, b)

        main_iters = _NB // nbuf

        def outer(j0, carry):
            for b in range(nbuf):
                j = j0 * nbuf + b
                gwait(j, b)

                @pl.when(j >= nbuf)
                def _():
                    swait(b)
                compute(b)
                sstart(j, b)

                @pl.when(j + nbuf < _NB)
                def _():
                    gstart(j + nbuf, b)
            return carry
        lax.fori_loop(0, main_iters, outer, 0)

        for j in range(main_iters * nbuf, _NB):
            b = j % nbuf
            gwait(j, b)
            swait(b)
            compute(b)
            sstart(j, b)
        for b in range(nbuf):
            swait(b)

        plsc.subcore_barrier()
        for k in range((nch + _NS - 1) // _NS):
            ch = sid + k * _NS

            @pl.when(ch < nch)
            def _():
                pltpu.sync_copy(acc.at[pl.ds(ch * zrows, zrows)], zbuf)
                pltpu.sync_copy(
                    zbuf, out_hbm.at[cid].at[pl.ds(ch * zrows, zrows)])

    return body


_SPECS1 = (("head", 0), ("head", 1), ("head", 2), ("head", 3), ("id", 0))


@functools.lru_cache(maxsize=None)
def _get_edge_pass(sw, aw, toff, idx_specs, nbuf):
    return _make_edge_pass(sw, aw, toff, idx_specs, nbuf)


@functools.lru_cache(maxsize=None)
def _get_edge_pass2():
    return _make_edge_pass2()


def kernel(x, edge_index, W1, att_src1, att_dst1, bias1,
           W2, att_src2, att_dst2, bias2):
    ei = edge_index.astype(jnp.int32)
    src = ei[0].reshape(_NW, _NB, _B)
    dst = ei[1].reshape(_NW, _NB, _B)

    f32 = jnp.float32
    heads = _HEADS
    oc = _HDIM // heads
    c64 = jnp.arange(_HDIM)
    j16 = jnp.arange(16)
    sel = (c64[:, None] // oc == j16[None, :]).astype(f32)
    As1 = sel * att_src1[0].reshape(-1)[:, None]
    Ad1 = sel * att_dst1[0].reshape(-1)[:, None]

    Rep = (jnp.arange(heads)[:, None] == (c64[None, :] // oc)).astype(f32)

    Q = jnp.zeros((2, 4), f32)
    Q = Q.at[0, 0].set(1.0).at[1, 1].set(1.0)
    Q = Q.at[0, 2].set(att_src2[0, 0, 0]).at[1, 2].set(att_src2[0, 0, 1])
    Q = Q.at[0, 3].set(att_dst2[0, 0, 0]).at[1, 3].set(att_dst2[0, 0, 1])
    W2Q = W2 @ Q

    b1 = bias1.reshape(1, _HDIM)
    b2 = bias2.reshape(1, 2)

    S1, D1, ms1, md1 = _prep1(x, W1, As1, Ad1)
    acc1 = _get_edge_pass(80, 80, 64, _SPECS1, 3)(
        S1, D1, ms1.reshape(16), md1.reshape(16), src, dst)
    P2, ms2, md2 = _prep2(acc1, Rep, b1, W2Q)
    acc2 = _get_edge_pass2()(
        P2, ms2.reshape(16), md2.reshape(16), src, dst)
    return _finish(acc2, b2).T
